# baseline (speedup 1.0000x reference)
import functools

import jax
import jax.numpy as jnp
from jax import lax
from jax.experimental import pallas as pl
from jax.experimental.pallas import tpu as pltpu
from jax.experimental.pallas import tpu_sc as plsc

B = 16384
VOCAB = 1000000
EMBED_DIM = 32
NUM_CONT = 13

_info = plsc.get_sparse_core_info()
NC, NS, L = _info.num_cores, _info.num_subcores, _info.num_lanes
NW = NC * NS
BPW = B // NW
NGRP = BPW // L

BLK = 65536
H = 5 * BLK
TB = VOCAB - H
_GRID = (TB + BLK - 1) // BLK
_OFF = H // BLK
BLKB = 2048
_ZSTEPS = B // BLKB

CPT = H // NW
CH = 1024
NCH = CPT // CH
NGC = CH // L

_mesh = plsc.VectorSubcoreMesh(core_axis_name="c", subcore_axis_name="s")


@functools.partial(
    pl.kernel,
    mesh=_mesh,
    out_type=jax.ShapeDtypeStruct((H,), jnp.float32),
    scratch_types=[
        pltpu.VMEM((EMBED_DIM, CH), jnp.float32),
        pltpu.VMEM((EMBED_DIM, CH), jnp.float32),
        pltpu.VMEM((CPT,), jnp.float32),
        pltpu.VMEM((2 * L,), jnp.float32),
        pltpu.SemaphoreType.DMA,
        pltpu.SemaphoreType.DMA,
    ],
    compiler_params=pltpu.CompilerParams(
        needs_layout_passes=False, skip_device_barrier=True),
)
def _sc_partial(t_hbm, w_hbm, yf_hbm, buf0, buf1, acc_v, w_v, sem0, sem1):
    wid = lax.axis_index("s") * NC + lax.axis_index("c")
    c0 = wid * CPT
    pltpu.sync_copy(w_hbm, w_v)
    wv = [w_v[pl.ds(k * L, L)] for k in range(2)]
    w = [wv[d // L][d % L] for d in range(EMBED_DIM)]
    bufs = (buf0, buf1)
    sems = (sem0, sem1)

    def start(ch):
        return pltpu.async_copy(
            t_hbm.at[pl.ds(0, EMBED_DIM), pl.ds(c0 + ch * CH, CH)],
            bufs[ch % 2], sems[ch % 2])

    cp = start(0)
    for ch in range(NCH):
        cp.wait()
        if ch + 1 < NCH:
            cp = start(ch + 1)
        buf = bufs[ch % 2]

        def g_body(g, carry, buf=buf, ch=ch):
            col = g * L
            acc = buf[0, pl.ds(col, L)] * w[0]
            for d in range(1, EMBED_DIM):
                acc = acc + buf[d, pl.ds(col, L)] * w[d]
            acc_v[pl.ds(ch * CH + col, L)] = acc
            return carry

        lax.fori_loop(0, NGC, g_body, 0)
    pltpu.sync_copy(acc_v, yf_hbm.at[pl.ds(c0, CPT)])


def _dense_body(t_ref, w_ref, x_ref, wcb_ref, y_ref, z_ref):
    i = pl.program_id(0)
    y_ref[...] = jax.lax.dot_general(
        w_ref[...], t_ref[...], (((0,), (0,)), ((), ())),
        preferred_element_type=jnp.float32)[0]

    @pl.when(i < _ZSTEPS)
    def _():
        z_ref[...] = jax.lax.dot_general(
            x_ref[...], wcb_ref[:NUM_CONT, :], (((1,), (0,)), ((), ())),
            preferred_element_type=jnp.float32)[:, 0] + wcb_ref[NUM_CONT, 0]


_dense = pl.pallas_call(
    _dense_body,
    grid=(_GRID,),
    in_specs=[
        pl.BlockSpec((EMBED_DIM, BLK), lambda i: (0, i + _OFF)),
        pl.BlockSpec((EMBED_DIM, 1), lambda i: (0, 0)),
        pl.BlockSpec((BLKB, NUM_CONT), lambda i: (jnp.minimum(i, _ZSTEPS - 1), 0)),
        pl.BlockSpec((NUM_CONT + 1, 1), lambda i: (0, 0)),
    ],
    out_specs=[
        pl.BlockSpec((BLK,), lambda i: (i,)),
        pl.BlockSpec((BLKB,), lambda i: (jnp.minimum(i, _ZSTEPS - 1),)),
    ],
    out_shape=[
        jax.ShapeDtypeStruct((TB,), jnp.float32),
        jax.ShapeDtypeStruct((B,), jnp.float32),
    ],
)


@functools.partial(
    pl.kernel,
    mesh=_mesh,
    out_type=jax.ShapeDtypeStruct((B,), jnp.float32),
    scratch_types=[
        pltpu.VMEM((BPW,), jnp.int32),
        pltpu.VMEM((BPW,), jnp.int32),
        pltpu.VMEM((BPW,), jnp.int32),
        pltpu.VMEM((BPW,), jnp.float32),
        pltpu.VMEM((BPW,), jnp.float32),
        pltpu.VMEM((BPW,), jnp.float32),
        pltpu.VMEM((BPW,), jnp.float32),
        pltpu.SemaphoreType.DMA,
        pltpu.SemaphoreType.DMA,
    ],
    compiler_params=pltpu.CompilerParams(
        needs_layout_passes=False, skip_device_barrier=True),
)
def _sc_lookup(idx_hbm, yf_hbm, yb_hbm, z_hbm, out_hbm,
               idx_v, idxf_v, idxb_v, yf_v, yb_v, z_v, out_v, semf, semb):
    wid = lax.axis_index("s") * NC + lax.axis_index("c")
    base = wid * BPW
    pltpu.sync_copy(idx_hbm.at[pl.ds(base, BPW)], idx_v)

    def split_body(g, carry):
        row0 = g * L
        iv = idx_v[pl.ds(row0, L)]
        idxf_v[pl.ds(row0, L)] = jnp.where(iv < H, iv, iv & 0x3FFFF)
        idxb_v[pl.ds(row0, L)] = jnp.where(iv >= H, iv - H, iv & 0x3FFFF)
        return carry

    lax.fori_loop(0, NGRP, split_body, 0)
    gf = pltpu.async_copy(yf_hbm.at[idxf_v], yf_v, semf)
    gb = pltpu.async_copy(yb_hbm.at[idxb_v], yb_v, semb)
    pltpu.sync_copy(z_hbm.at[pl.ds(base, BPW)], z_v)
    gf.wait()
    gb.wait()

    def body(g, carry):
        row0 = g * L
        sel = idx_v[pl.ds(row0, L)] < H
        y16 = jnp.where(sel, yf_v[pl.ds(row0, L)], yb_v[pl.ds(row0, L)])
        out_v[pl.ds(row0, L)] = y16 + z_v[pl.ds(row0, L)]
        return carry

    lax.fori_loop(0, NGRP, body, 0)
    pltpu.sync_copy(out_v, out_hbm.at[pl.ds(base, BPW)])


def kernel(x_cat, x_cont, emb_table, fc_w, fc_b):
    table_t = emb_table.T
    w_col = fc_w[:EMBED_DIM]
    w_flat = fc_w[:EMBED_DIM, 0]
    wcb = jnp.concatenate([fc_w[EMBED_DIM:, 0], fc_b]).reshape(NUM_CONT + 1, 1)
    y_front = _sc_partial(table_t, w_flat)
    y_back, z = _dense(table_t, w_col, x_cont, wcb)
    idx = x_cat.reshape(B)
    out = _sc_lookup(idx, y_front, y_back, z)
    return out.reshape(B, 1)


# speedup vs baseline: 1.0055x; 1.0055x over previous
import functools

import jax
import jax.numpy as jnp
from jax import lax
from jax.experimental import pallas as pl
from jax.experimental.pallas import tpu as pltpu
from jax.experimental.pallas import tpu_sc as plsc

B = 16384
VOCAB = 1000000
EMBED_DIM = 32
NUM_CONT = 13

_info = plsc.get_sparse_core_info()
NC, NS, L = _info.num_cores, _info.num_subcores, _info.num_lanes
NW = NC * NS
BPW = B // NW
NGRP = BPW // L

BLK = 65536
H = 4 * BLK
TB = VOCAB - H
_GRID = (TB + BLK - 1) // BLK
_OFF = H // BLK
BLKB = 2048
_ZSTEPS = B // BLKB

CPT = H // NW
CH = 1024
NCH = CPT // CH
NGC = CH // L

_mesh = plsc.VectorSubcoreMesh(core_axis_name="c", subcore_axis_name="s")


@functools.partial(
    pl.kernel,
    mesh=_mesh,
    out_type=jax.ShapeDtypeStruct((H,), jnp.float32),
    scratch_types=[
        pltpu.VMEM((EMBED_DIM, CH), jnp.float32),
        pltpu.VMEM((EMBED_DIM, CH), jnp.float32),
        pltpu.VMEM((CPT,), jnp.float32),
        pltpu.VMEM((2 * L,), jnp.float32),
        pltpu.SemaphoreType.DMA,
        pltpu.SemaphoreType.DMA,
    ],
    compiler_params=pltpu.CompilerParams(
        needs_layout_passes=False, skip_device_barrier=True),
)
def _sc_partial(t_hbm, w_hbm, yf_hbm, buf0, buf1, acc_v, w_v, sem0, sem1):
    wid = lax.axis_index("s") * NC + lax.axis_index("c")
    c0 = wid * CPT
    pltpu.sync_copy(w_hbm, w_v)
    wv = [w_v[pl.ds(k * L, L)] for k in range(2)]
    w = [wv[d // L][d % L] for d in range(EMBED_DIM)]
    bufs = (buf0, buf1)
    sems = (sem0, sem1)

    def start(ch):
        return pltpu.async_copy(
            t_hbm.at[pl.ds(0, EMBED_DIM), pl.ds(c0 + ch * CH, CH)],
            bufs[ch % 2], sems[ch % 2])

    cp = start(0)
    for ch in range(NCH):
        cp.wait()
        if ch + 1 < NCH:
            cp = start(ch + 1)
        buf = bufs[ch % 2]

        def g_body(g, carry, buf=buf, ch=ch):
            col = g * L
            acc = buf[0, pl.ds(col, L)] * w[0]
            for d in range(1, EMBED_DIM):
                acc = acc + buf[d, pl.ds(col, L)] * w[d]
            acc_v[pl.ds(ch * CH + col, L)] = acc
            return carry

        lax.fori_loop(0, NGC, g_body, 0)
    pltpu.sync_copy(acc_v, yf_hbm.at[pl.ds(c0, CPT)])


def _dense_body(t_ref, w_ref, x_ref, wcb_ref, y_ref, z_ref):
    i = pl.program_id(0)
    y_ref[...] = jax.lax.dot_general(
        w_ref[...], t_ref[...], (((0,), (0,)), ((), ())),
        preferred_element_type=jnp.float32)[0]

    @pl.when(i < _ZSTEPS)
    def _():
        z_ref[...] = jax.lax.dot_general(
            x_ref[...], wcb_ref[:NUM_CONT, :], (((1,), (0,)), ((), ())),
            preferred_element_type=jnp.float32)[:, 0] + wcb_ref[NUM_CONT, 0]


_dense = pl.pallas_call(
    _dense_body,
    grid=(_GRID,),
    in_specs=[
        pl.BlockSpec((EMBED_DIM, BLK), lambda i: (0, i + _OFF)),
        pl.BlockSpec((EMBED_DIM, 1), lambda i: (0, 0)),
        pl.BlockSpec((BLKB, NUM_CONT), lambda i: (jnp.minimum(i, _ZSTEPS - 1), 0)),
        pl.BlockSpec((NUM_CONT + 1, 1), lambda i: (0, 0)),
    ],
    out_specs=[
        pl.BlockSpec((BLK,), lambda i: (i,)),
        pl.BlockSpec((BLKB,), lambda i: (jnp.minimum(i, _ZSTEPS - 1),)),
    ],
    out_shape=[
        jax.ShapeDtypeStruct((TB,), jnp.float32),
        jax.ShapeDtypeStruct((B,), jnp.float32),
    ],
)


@functools.partial(
    pl.kernel,
    mesh=_mesh,
    out_type=jax.ShapeDtypeStruct((B,), jnp.float32),
    scratch_types=[
        pltpu.VMEM((BPW,), jnp.int32),
        pltpu.VMEM((BPW,), jnp.int32),
        pltpu.VMEM((BPW,), jnp.int32),
        pltpu.VMEM((BPW,), jnp.float32),
        pltpu.VMEM((BPW,), jnp.float32),
        pltpu.VMEM((BPW,), jnp.float32),
        pltpu.VMEM((BPW,), jnp.float32),
        pltpu.SemaphoreType.DMA,
        pltpu.SemaphoreType.DMA,
    ],
    compiler_params=pltpu.CompilerParams(
        needs_layout_passes=False, skip_device_barrier=True),
)
def _sc_lookup(idx_hbm, yf_hbm, yb_hbm, z_hbm, out_hbm,
               idx_v, idxf_v, idxb_v, yf_v, yb_v, z_v, out_v, semf, semb):
    wid = lax.axis_index("s") * NC + lax.axis_index("c")
    base = wid * BPW
    pltpu.sync_copy(idx_hbm.at[pl.ds(base, BPW)], idx_v)

    def split_body(g, carry):
        row0 = g * L
        iv = idx_v[pl.ds(row0, L)]
        idxf_v[pl.ds(row0, L)] = jnp.where(iv < H, iv, iv & 0x3FFFF)
        idxb_v[pl.ds(row0, L)] = jnp.where(iv >= H, iv - H, iv & 0x3FFFF)
        return carry

    lax.fori_loop(0, NGRP, split_body, 0)
    gf = pltpu.async_copy(yf_hbm.at[idxf_v], yf_v, semf)
    gb = pltpu.async_copy(yb_hbm.at[idxb_v], yb_v, semb)
    pltpu.sync_copy(z_hbm.at[pl.ds(base, BPW)], z_v)
    gf.wait()
    gb.wait()

    def body(g, carry):
        row0 = g * L
        sel = idx_v[pl.ds(row0, L)] < H
        y16 = jnp.where(sel, yf_v[pl.ds(row0, L)], yb_v[pl.ds(row0, L)])
        out_v[pl.ds(row0, L)] = y16 + z_v[pl.ds(row0, L)]
        return carry

    lax.fori_loop(0, NGRP, body, 0)
    pltpu.sync_copy(out_v, out_hbm.at[pl.ds(base, BPW)])


def kernel(x_cat, x_cont, emb_table, fc_w, fc_b):
    table_t = emb_table.T
    w_col = fc_w[:EMBED_DIM]
    w_flat = fc_w[:EMBED_DIM, 0]
    wcb = jnp.concatenate([fc_w[EMBED_DIM:, 0], fc_b]).reshape(NUM_CONT + 1, 1)
    y_front = _sc_partial(table_t, w_flat)
    y_back, z = _dense(table_t, w_col, x_cont, wcb)
    idx = x_cat.reshape(B)
    out = _sc_lookup(idx, y_front, y_back, z)
    return out.reshape(B, 1)
